# Initial kernel scaffold; baseline (speedup 1.0000x reference)
import jax
import jax.numpy as jnp
from jax.experimental import pallas as pl


def kernel(cut_positions, cut_local_reflatentxgene_ix, cut_local_gene_ix, cut_local_reflatent_ix, mixture_delta_reflatentxgene, unnormalized_heights, unnormalized_widths):
    raise NotImplementedError("write your pallas kernel here")


# speedup vs baseline: 17.2545x; 17.2545x over previous
import jax
import jax.numpy as jnp
from jax import lax
from jax.experimental import pallas as pl
from jax.experimental.pallas import tpu as pltpu
from jax.experimental.pallas import tpu_sc as plsc

N_GENES = 128
N_REFLATENT = 16
N_CUTS = 2048
B1 = 64
B2 = 32
W1 = 4 * B1
W2 = 4 * B2

_NC = 2
_NS = 16
_NW = _NC * _NS
_CPW = N_CUTS // _NW

_HIGH = lax.Precision.HIGHEST


def _strict_lower(n):
    ii = lax.broadcasted_iota(jnp.int32, (n, n), 0)
    jj = lax.broadcasted_iota(jnp.int32, (n, n), 1)
    return (ii < jj).astype(jnp.float32)


def _shift_down_genes(x, k):
    pad = jnp.zeros((N_REFLATENT, k, 1), jnp.float32)
    return jnp.concatenate([pad, x[:, : N_GENES - k, :]], axis=1)


def _excl_cumsum_genes(s):
    incl = s
    k = 1
    while k < N_GENES:
        incl = incl + _shift_down_genes(incl, k)
        k *= 2
    return incl - s


def _tables_body(uw1_ref, uw2_ref, uh1_ref, uh2_ref, d1_ref, d2_ref,
                 o1_ref, o2_ref):
    f32 = jnp.float32
    M64 = _strict_lower(B1)
    M32 = _strict_lower(B2)

    def transform(uw, uh, d, gspace, nb, M, o_ref):
        e = jnp.exp(uw)
        sm = e / jnp.sum(e, axis=-1, keepdims=True)
        w3 = sm[None, :, :] * gspace
        cum_in = jnp.dot(w3.reshape(N_REFLATENT * N_GENES, nb), M,
                         precision=_HIGH).reshape(N_REFLATENT, N_GENES, nb)
        s_w = jnp.sum(w3, axis=-1, keepdims=True)
        offs_w = _excl_cumsum_genes(s_w)
        gg = lax.broadcasted_iota(jnp.int32, (N_REFLATENT, N_GENES, nb), 1)
        bb = lax.broadcasted_iota(jnp.int32, (N_REFLATENT, N_GENES, nb), 2)
        last = jnp.logical_and(gg == N_GENES - 1, bb == nb - 1)
        locs = jnp.where(last, f32(1.0), offs_w + cum_in)
        o_ref[:, :, 0:nb] = locs
        o_ref[:, :, nb:2 * nb] = w3
        h = jnp.exp(d + uh[None, :, :])
        hn = jnp.concatenate([h[:, :, 1:], h[:, :, nb - 1:]], axis=-1)
        c = (h + hn) * 0.5 * w3
        s_c = jnp.sum(c, axis=-1, keepdims=True)
        area = jnp.sum(s_c, axis=1, keepdims=True)
        inv_area = 1.0 / area
        o_ref[:, :, 2 * nb:3 * nb] = h * inv_area
        cn = c * inv_area
        cdf_in = jnp.dot(cn.reshape(N_REFLATENT * N_GENES, nb), M,
                         precision=_HIGH).reshape(N_REFLATENT, N_GENES, nb)
        s_cn = s_c * inv_area
        offs_c = _excl_cumsum_genes(s_cn)
        cdf = jnp.where(last, f32(1.0), offs_c + cdf_in)
        o_ref[:, :, 3 * nb:4 * nb] = cdf
        gs = offs_c + s_cn
        gg1 = lax.broadcasted_iota(jnp.int32, (N_REFLATENT, N_GENES, 1), 1)
        gs = jnp.where(gg1 == N_GENES - 1, f32(1.0), gs)
        return gs - _shift_down_genes(gs, 1)

    gs1 = jnp.full((N_REFLATENT, N_GENES, 1), 1.0 / N_GENES, jnp.float32)
    gs2 = transform(uw1_ref[...], uh1_ref[...], d1_ref[...], gs1, B1, M64,
                    o1_ref)
    transform(uw2_ref[...], uh2_ref[...], d2_ref[...], gs2, B2, M32, o2_ref)


def _compute_tables(uw1p, uw2p, uh1, uh2, d1, d2):
    f32 = jnp.float32
    sds = jax.ShapeDtypeStruct
    out_shape = [
        sds((N_REFLATENT, N_GENES, W1), f32),
        sds((N_REFLATENT, N_GENES, W2), f32),
    ]
    return pl.pallas_call(_tables_body, out_shape=out_shape)(
        uw1p, uw2p, uh1, uh2, d1, d2)


def _spline_batch(tab, rows, x, nb):
    i32 = jnp.int32
    lo = jnp.zeros((16,), i32)
    step = nb // 2
    while step >= 1:
        edge = plsc.load_gather(tab, [rows, lo + (step - 1)])
        lo = lo + jnp.where(edge < x, i32(step), i32(0))
        step //= 2
    jr = jnp.minimum(jnp.maximum(lo - 1, i32(0)), i32(nb - 2))
    ll = plsc.load_gather(tab, [rows, jr])
    w = plsc.load_gather(tab, [rows, nb + jr])
    hl = plsc.load_gather(tab, [rows, 2 * nb + jr])
    hr = plsc.load_gather(tab, [rows, 2 * nb + 1 + jr])
    cl = plsc.load_gather(tab, [rows, 3 * nb + jr])
    alpha = (x - ll) / w
    dh = hr - hl
    out = (0.5 * dh * w) * alpha * alpha + (hl * w) * alpha + cl
    out = jnp.minimum(jnp.maximum(out, 0.0), 1.0)
    hval = hl + alpha * dh
    return out, hval


def _sc_body(x_hbm, rxg_hbm, t1, t2, out_hbm, hp_hbm,
             idxv, xbuf, hp, outb, tab1, tab2, sem1, sem2):
    wid = lax.axis_index("s") * _NC + lax.axis_index("c")
    base = wid * _CPW
    pltpu.sync_copy(rxg_hbm.at[pl.ds(base, _CPW)], idxv)
    pltpu.sync_copy(x_hbm.at[pl.ds(base, _CPW)], xbuf)
    cp1 = pltpu.async_copy(t1.at[idxv], tab1, sem1)
    cp2 = pltpu.async_copy(t2.at[idxv], tab2, sem2)
    cp1.wait()
    iota = lax.broadcasted_iota(jnp.int32, (16,), 0)
    for v in range(_CPW // 16):
        sl = pl.ds(v * 16, 16)
        rows = iota + (v * 16)
        out, hval = _spline_batch(tab1, rows, xbuf[sl], B1)
        xbuf[sl] = out
        hp[sl] = hval
    cp2.wait()
    for v in range(_CPW // 16):
        sl = pl.ds(v * 16, 16)
        rows = iota + (v * 16)
        out, hval = _spline_batch(tab2, rows, xbuf[sl], B2)
        outb[sl] = out
        hp[sl] = hp[sl] * hval
    pltpu.sync_copy(outb, out_hbm.at[pl.ds(base, _CPW)])
    pltpu.sync_copy(hp, hp_hbm.at[pl.ds(base, _CPW)])


def _sc_transform(x, rxg, t1, t2):
    f32 = jnp.float32
    sds = jax.ShapeDtypeStruct
    mesh = plsc.VectorSubcoreMesh(core_axis_name="c", subcore_axis_name="s",
                                  num_cores=_NC, num_subcores=_NS)
    scratch = [
        pltpu.VMEM((_CPW,), jnp.int32),
        pltpu.VMEM((_CPW,), f32),
        pltpu.VMEM((_CPW,), f32),
        pltpu.VMEM((_CPW,), f32),
        pltpu.VMEM((_CPW, W1), f32),
        pltpu.VMEM((_CPW, W2), f32),
        pltpu.SemaphoreType.DMA,
        pltpu.SemaphoreType.DMA,
    ]
    fn = pl.kernel(_sc_body,
                   out_type=(sds((N_CUTS,), f32), sds((N_CUTS,), f32)),
                   mesh=mesh, scratch_types=scratch,
                   compiler_params=pltpu.CompilerParams(
                       needs_layout_passes=False))
    return fn(x, rxg, t1, t2)


def _log_body(x_ref, o_ref):
    o_ref[...] = jnp.log(x_ref[...])


def _log_call(hprod):
    r = pl.pallas_call(
        _log_body,
        out_shape=jax.ShapeDtypeStruct((N_REFLATENT, N_CUTS // N_REFLATENT),
                                       jnp.float32),
    )(hprod.reshape(N_REFLATENT, N_CUTS // N_REFLATENT))
    return r.reshape(N_CUTS)


def kernel(cut_positions, cut_local_reflatentxgene_ix, cut_local_gene_ix,
           cut_local_reflatent_ix, mixture_delta_reflatentxgene,
           unnormalized_heights, unnormalized_widths):
    del cut_local_gene_ix, cut_local_reflatent_ix
    uw = unnormalized_widths
    uh = unnormalized_heights
    neg = jnp.float32(-1e9)
    uw1p = jnp.pad(uw[:, : B1 - 1], ((0, 0), (0, 1)), constant_values=neg)
    uw2p = jnp.pad(uw[:, B1 - 1:], ((0, 0), (0, 1)), constant_values=neg)
    uh1 = uh[:, :B1]
    uh2 = uh[:, B1:]
    d1 = mixture_delta_reflatentxgene[:, :, :B1]
    d2 = mixture_delta_reflatentxgene[:, :, B1:]
    t1, t2 = _compute_tables(uw1p, uw2p, uh1, uh2, d1, d2)
    t1 = t1.reshape(N_REFLATENT * N_GENES, W1)
    t2 = t2.reshape(N_REFLATENT * N_GENES, W2)
    rxg = cut_local_reflatentxgene_ix.astype(jnp.int32)
    out, hprod = _sc_transform(cut_positions, rxg, t1, t2)
    logabsdet = _log_call(hprod)
    return out, logabsdet


# speedup vs baseline: 17.7988x; 1.0315x over previous
import jax
import jax.numpy as jnp
from jax import lax
from jax.experimental import pallas as pl
from jax.experimental.pallas import tpu as pltpu
from jax.experimental.pallas import tpu_sc as plsc

N_GENES = 128
N_REFLATENT = 16
N_CUTS = 2048
B1 = 64
B2 = 32
W1 = 4 * B1
W2 = 4 * B2

_NC = 2
_NS = 16
_NW = _NC * _NS
_CPW = N_CUTS // _NW

_HIGH = lax.Precision.HIGHEST


def _strict_lower(n):
    ii = lax.broadcasted_iota(jnp.int32, (n, n), 0)
    jj = lax.broadcasted_iota(jnp.int32, (n, n), 1)
    return (ii < jj).astype(jnp.float32)


def _shift_down_genes(x, k):
    pad = jnp.zeros((N_REFLATENT, k, 1), jnp.float32)
    return jnp.concatenate([pad, x[:, : N_GENES - k, :]], axis=1)


def _excl_cumsum_genes(s):
    incl = s
    k = 1
    while k < N_GENES:
        incl = incl + _shift_down_genes(incl, k)
        k *= 2
    return incl - s


def _tables_body(uw1_ref, uw2_ref, uh1_ref, uh2_ref, d1_ref, d2_ref,
                 o1_ref, o2_ref):
    f32 = jnp.float32
    M64 = _strict_lower(B1)
    M32 = _strict_lower(B2)

    def transform(uw, uh, d, gspace, nb, M, o_ref):
        e = jnp.exp(uw)
        sm = e / jnp.sum(e, axis=-1, keepdims=True)
        w3 = sm[None, :, :] * gspace
        cum_in = jnp.dot(w3.reshape(N_REFLATENT * N_GENES, nb), M,
                         precision=_HIGH).reshape(N_REFLATENT, N_GENES, nb)
        s_w = jnp.sum(w3, axis=-1, keepdims=True)
        offs_w = _excl_cumsum_genes(s_w)
        gg = lax.broadcasted_iota(jnp.int32, (N_REFLATENT, N_GENES, nb), 1)
        bb = lax.broadcasted_iota(jnp.int32, (N_REFLATENT, N_GENES, nb), 2)
        last = jnp.logical_and(gg == N_GENES - 1, bb == nb - 1)
        locs = jnp.where(last, f32(1.0), offs_w + cum_in)
        o_ref[:, :, 0:nb] = locs
        o_ref[:, :, nb:2 * nb] = w3
        h = jnp.exp(d + uh[None, :, :])
        hn = jnp.concatenate([h[:, :, 1:], h[:, :, nb - 1:]], axis=-1)
        c = (h + hn) * 0.5 * w3
        s_c = jnp.sum(c, axis=-1, keepdims=True)
        area = jnp.sum(s_c, axis=1, keepdims=True)
        inv_area = 1.0 / area
        o_ref[:, :, 2 * nb:3 * nb] = h * inv_area
        cn = c * inv_area
        cdf_in = jnp.dot(cn.reshape(N_REFLATENT * N_GENES, nb), M,
                         precision=_HIGH).reshape(N_REFLATENT, N_GENES, nb)
        s_cn = s_c * inv_area
        offs_c = _excl_cumsum_genes(s_cn)
        cdf = jnp.where(last, f32(1.0), offs_c + cdf_in)
        o_ref[:, :, 3 * nb:4 * nb] = cdf
        gs = offs_c + s_cn
        gg1 = lax.broadcasted_iota(jnp.int32, (N_REFLATENT, N_GENES, 1), 1)
        gs = jnp.where(gg1 == N_GENES - 1, f32(1.0), gs)
        return gs - _shift_down_genes(gs, 1)

    gs1 = jnp.full((N_REFLATENT, N_GENES, 1), 1.0 / N_GENES, jnp.float32)
    gs2 = transform(uw1_ref[...], uh1_ref[...], d1_ref[...], gs1, B1, M64,
                    o1_ref)
    transform(uw2_ref[...], uh2_ref[...], d2_ref[...], gs2, B2, M32, o2_ref)


def _compute_tables(uw1p, uw2p, uh1, uh2, d1, d2):
    f32 = jnp.float32
    sds = jax.ShapeDtypeStruct
    out_shape = [
        sds((N_REFLATENT, N_GENES, W1), f32),
        sds((N_REFLATENT, N_GENES, W2), f32),
    ]
    return pl.pallas_call(_tables_body, out_shape=out_shape)(
        uw1p, uw2p, uh1, uh2, d1, d2)


def _spline_batch(tab, rows, x, nb):
    i32 = jnp.int32
    lo = jnp.zeros((16,), i32)
    step = nb // 2
    while step >= 1:
        edge = plsc.load_gather(tab, [rows, lo + (step - 1)])
        lo = lo + jnp.where(edge < x, i32(step), i32(0))
        step //= 2
    jr = jnp.minimum(jnp.maximum(lo - 1, i32(0)), i32(nb - 2))
    ll = plsc.load_gather(tab, [rows, jr])
    w = plsc.load_gather(tab, [rows, nb + jr])
    hl = plsc.load_gather(tab, [rows, 2 * nb + jr])
    hr = plsc.load_gather(tab, [rows, 2 * nb + 1 + jr])
    cl = plsc.load_gather(tab, [rows, 3 * nb + jr])
    alpha = (x - ll) / w
    dh = hr - hl
    out = (0.5 * dh * w) * alpha * alpha + (hl * w) * alpha + cl
    out = jnp.minimum(jnp.maximum(out, 0.0), 1.0)
    hval = hl + alpha * dh
    return out, hval


def _ln16(x):
    i32 = jnp.int32
    bits = plsc.bitcast(x, jnp.int32)
    e = lax.shift_right_logical(bits, 23) - 127
    m = plsc.bitcast((bits & i32(0x7FFFFF)) | i32(127 << 23), jnp.float32)
    big = m > 1.4142135
    m = jnp.where(big, m * 0.5, m)
    e = e + jnp.where(big, i32(1), i32(0))
    r = (m - 1.0) / (m + 1.0)
    r2 = r * r
    p = 2.0 * r * (1.0 + r2 * (1.0 / 3.0 + r2 * (0.2 + r2 * (1.0 / 7.0))))
    return e.astype(jnp.float32) * 0.6931471805599453 + p


def _sc_body(x_hbm, rxg_hbm, t1, t2, out_hbm, lad_hbm,
             idxv, xbuf, hp, outb, tab1, tab2, sem1, sem2):
    wid = lax.axis_index("s") * _NC + lax.axis_index("c")
    base = wid * _CPW
    pltpu.sync_copy(rxg_hbm.at[pl.ds(base, _CPW)], idxv)
    pltpu.sync_copy(x_hbm.at[pl.ds(base, _CPW)], xbuf)
    cp1 = pltpu.async_copy(t1.at[idxv], tab1, sem1)
    cp2 = pltpu.async_copy(t2.at[idxv], tab2, sem2)
    cp1.wait()
    iota = lax.broadcasted_iota(jnp.int32, (16,), 0)
    for v in range(_CPW // 16):
        sl = pl.ds(v * 16, 16)
        rows = iota + (v * 16)
        out, hval = _spline_batch(tab1, rows, xbuf[sl], B1)
        xbuf[sl] = out
        hp[sl] = hval
    cp2.wait()
    for v in range(_CPW // 16):
        sl = pl.ds(v * 16, 16)
        rows = iota + (v * 16)
        out, hval = _spline_batch(tab2, rows, xbuf[sl], B2)
        outb[sl] = out
        hp[sl] = _ln16(hp[sl] * hval)
    pltpu.sync_copy(outb, out_hbm.at[pl.ds(base, _CPW)])
    pltpu.sync_copy(hp, lad_hbm.at[pl.ds(base, _CPW)])


def _sc_transform(x, rxg, t1, t2):
    f32 = jnp.float32
    sds = jax.ShapeDtypeStruct
    mesh = plsc.VectorSubcoreMesh(core_axis_name="c", subcore_axis_name="s",
                                  num_cores=_NC, num_subcores=_NS)
    scratch = [
        pltpu.VMEM((_CPW,), jnp.int32),
        pltpu.VMEM((_CPW,), f32),
        pltpu.VMEM((_CPW,), f32),
        pltpu.VMEM((_CPW,), f32),
        pltpu.VMEM((_CPW, W1), f32),
        pltpu.VMEM((_CPW, W2), f32),
        pltpu.SemaphoreType.DMA,
        pltpu.SemaphoreType.DMA,
    ]
    fn = pl.kernel(_sc_body,
                   out_type=(sds((N_CUTS,), f32), sds((N_CUTS,), f32)),
                   mesh=mesh, scratch_types=scratch,
                   compiler_params=pltpu.CompilerParams(
                       needs_layout_passes=False))
    return fn(x, rxg, t1, t2)


def kernel(cut_positions, cut_local_reflatentxgene_ix, cut_local_gene_ix,
           cut_local_reflatent_ix, mixture_delta_reflatentxgene,
           unnormalized_heights, unnormalized_widths):
    del cut_local_gene_ix, cut_local_reflatent_ix
    uw = unnormalized_widths
    uh = unnormalized_heights
    neg = jnp.float32(-1e9)
    uw1p = jnp.pad(uw[:, : B1 - 1], ((0, 0), (0, 1)), constant_values=neg)
    uw2p = jnp.pad(uw[:, B1 - 1:], ((0, 0), (0, 1)), constant_values=neg)
    uh1 = uh[:, :B1]
    uh2 = uh[:, B1:]
    d1 = mixture_delta_reflatentxgene[:, :, :B1]
    d2 = mixture_delta_reflatentxgene[:, :, B1:]
    t1, t2 = _compute_tables(uw1p, uw2p, uh1, uh2, d1, d2)
    t1 = t1.reshape(N_REFLATENT * N_GENES, W1)
    t2 = t2.reshape(N_REFLATENT * N_GENES, W2)
    rxg = cut_local_reflatentxgene_ix.astype(jnp.int32)
    out, logabsdet = _sc_transform(cut_positions, rxg, t1, t2)
    return out, logabsdet


# speedup vs baseline: 18.3592x; 1.0315x over previous
import jax
import jax.numpy as jnp
from jax import lax
from jax.experimental import pallas as pl
from jax.experimental.pallas import tpu as pltpu
from jax.experimental.pallas import tpu_sc as plsc

N_GENES = 128
N_REFLATENT = 16
N_CUTS = 2048
B1 = 64
B2 = 32
W1 = 4 * B1
W2 = 4 * B2

_NC = 2
_NS = 16
_NW = _NC * _NS
_CPW = N_CUTS // _NW

_HIGH = lax.Precision.HIGHEST


def _strict_lower(n):
    ii = lax.broadcasted_iota(jnp.int32, (n, n), 0)
    jj = lax.broadcasted_iota(jnp.int32, (n, n), 1)
    return (ii < jj).astype(jnp.float32)


def _shift_down_genes(x, k):
    pad = jnp.zeros((N_REFLATENT, k, 1), jnp.float32)
    return jnp.concatenate([pad, x[:, : N_GENES - k, :]], axis=1)


def _excl_cumsum_genes(s, mg):
    del mg
    incl = s
    k = 1
    while k < N_GENES:
        incl = incl + _shift_down_genes(incl, k)
        k *= 2
    return incl - s


def _tables_body(uw_ref, uh_ref, d_ref, o1_ref, o2_ref):
    f32 = jnp.float32
    M64 = _strict_lower(B1)
    M32 = _strict_lower(B2)
    MG = _strict_lower(N_GENES)
    neg = jnp.full((N_GENES, 1), -1e9, f32)
    uw_all = uw_ref[...]
    uh_all = uh_ref[...]
    d_all = d_ref[...]
    uw1 = jnp.concatenate([uw_all[:, : B1 - 1], neg], axis=-1)
    uw2 = jnp.concatenate([uw_all[:, B1 - 1:], neg], axis=-1)
    uh1 = uh_all[:, :B1]
    uh2 = uh_all[:, B1:]
    d1 = d_all[:, :, :B1]
    d2 = d_all[:, :, B1:]

    def transform(uw, uh, d, gspace, nb, M, o_ref):
        e = jnp.exp(uw)
        sm = e / jnp.sum(e, axis=-1, keepdims=True)
        w3 = sm[None, :, :] * gspace
        cum_in = jnp.dot(w3.reshape(N_REFLATENT * N_GENES, nb), M,
                         precision=_HIGH).reshape(N_REFLATENT, N_GENES, nb)
        s_w = jnp.sum(w3, axis=-1, keepdims=True)
        offs_w = _excl_cumsum_genes(s_w, MG)
        gg = lax.broadcasted_iota(jnp.int32, (N_REFLATENT, N_GENES, nb), 1)
        bb = lax.broadcasted_iota(jnp.int32, (N_REFLATENT, N_GENES, nb), 2)
        last = jnp.logical_and(gg == N_GENES - 1, bb == nb - 1)
        locs = jnp.where(last, f32(1.0), offs_w + cum_in)
        o_ref[:, :, 0:nb] = locs
        o_ref[:, :, nb:2 * nb] = w3
        h = jnp.exp(d + uh[None, :, :])
        hn = jnp.concatenate([h[:, :, 1:], h[:, :, nb - 1:]], axis=-1)
        c = (h + hn) * 0.5 * w3
        s_c = jnp.sum(c, axis=-1, keepdims=True)
        area = jnp.sum(s_c, axis=1, keepdims=True)
        inv_area = 1.0 / area
        o_ref[:, :, 2 * nb:3 * nb] = h * inv_area
        cn = c * inv_area
        cdf_in = jnp.dot(cn.reshape(N_REFLATENT * N_GENES, nb), M,
                         precision=_HIGH).reshape(N_REFLATENT, N_GENES, nb)
        s_cn = s_c * inv_area
        offs_c = _excl_cumsum_genes(s_cn, MG)
        cdf = jnp.where(last, f32(1.0), offs_c + cdf_in)
        o_ref[:, :, 3 * nb:4 * nb] = cdf
        gs = offs_c + s_cn
        gg1 = lax.broadcasted_iota(jnp.int32, (N_REFLATENT, N_GENES, 1), 1)
        gs = jnp.where(gg1 == N_GENES - 1, f32(1.0), gs)
        return gs - _shift_down_genes(gs, 1)

    gs1 = jnp.full((N_REFLATENT, N_GENES, 1), 1.0 / N_GENES, jnp.float32)
    gs2 = transform(uw1, uh1, d1, gs1, B1, M64, o1_ref)
    transform(uw2, uh2, d2, gs2, B2, M32, o2_ref)


def _compute_tables(uw, uh, d):
    f32 = jnp.float32
    sds = jax.ShapeDtypeStruct
    out_shape = [
        sds((N_REFLATENT, N_GENES, W1), f32),
        sds((N_REFLATENT, N_GENES, W2), f32),
    ]
    return pl.pallas_call(_tables_body, out_shape=out_shape)(uw, uh, d)


def _spline_batch(tab, rows, x, nb):
    i32 = jnp.int32
    lo = jnp.zeros((16,), i32)
    step = nb // 2
    while step >= 1:
        edge = plsc.load_gather(tab, [rows, lo + (step - 1)])
        lo = lo + jnp.where(edge < x, i32(step), i32(0))
        step //= 2
    jr = jnp.minimum(jnp.maximum(lo - 1, i32(0)), i32(nb - 2))
    ll = plsc.load_gather(tab, [rows, jr])
    w = plsc.load_gather(tab, [rows, nb + jr])
    hl = plsc.load_gather(tab, [rows, 2 * nb + jr])
    hr = plsc.load_gather(tab, [rows, 2 * nb + 1 + jr])
    cl = plsc.load_gather(tab, [rows, 3 * nb + jr])
    alpha = (x - ll) / w
    dh = hr - hl
    out = (0.5 * dh * w) * alpha * alpha + (hl * w) * alpha + cl
    out = jnp.minimum(jnp.maximum(out, 0.0), 1.0)
    hval = hl + alpha * dh
    return out, hval


def _ln16(x):
    i32 = jnp.int32
    bits = plsc.bitcast(x, jnp.int32)
    e = lax.shift_right_logical(bits, 23) - 127
    m = plsc.bitcast((bits & i32(0x7FFFFF)) | i32(127 << 23), jnp.float32)
    big = m > 1.4142135
    m = jnp.where(big, m * 0.5, m)
    e = e + jnp.where(big, i32(1), i32(0))
    r = (m - 1.0) / (m + 1.0)
    r2 = r * r
    p = 2.0 * r * (1.0 + r2 * (1.0 / 3.0 + r2 * (0.2 + r2 * (1.0 / 7.0))))
    return e.astype(jnp.float32) * 0.6931471805599453 + p


def _sc_body(x_hbm, rxg_hbm, t1, t2, out_hbm, lad_hbm,
             idxv, xbuf, hp, outb, tab1, tab2, sem1, sem2):
    wid = lax.axis_index("s") * _NC + lax.axis_index("c")
    base = wid * _CPW
    pltpu.sync_copy(rxg_hbm.at[pl.ds(base, _CPW)], idxv)
    pltpu.sync_copy(x_hbm.at[pl.ds(base, _CPW)], xbuf)
    cp1 = pltpu.async_copy(t1.at[idxv], tab1, sem1)
    cp2 = pltpu.async_copy(t2.at[idxv], tab2, sem2)
    cp1.wait()
    iota = lax.broadcasted_iota(jnp.int32, (16,), 0)
    for v in range(_CPW // 16):
        sl = pl.ds(v * 16, 16)
        rows = iota + (v * 16)
        out, hval = _spline_batch(tab1, rows, xbuf[sl], B1)
        xbuf[sl] = out
        hp[sl] = hval
    cp2.wait()
    for v in range(_CPW // 16):
        sl = pl.ds(v * 16, 16)
        rows = iota + (v * 16)
        out, hval = _spline_batch(tab2, rows, xbuf[sl], B2)
        outb[sl] = out
        hp[sl] = _ln16(hp[sl] * hval)
    pltpu.sync_copy(outb, out_hbm.at[pl.ds(base, _CPW)])
    pltpu.sync_copy(hp, lad_hbm.at[pl.ds(base, _CPW)])


def _sc_transform(x, rxg, t1, t2):
    f32 = jnp.float32
    sds = jax.ShapeDtypeStruct
    mesh = plsc.VectorSubcoreMesh(core_axis_name="c", subcore_axis_name="s",
                                  num_cores=_NC, num_subcores=_NS)
    scratch = [
        pltpu.VMEM((_CPW,), jnp.int32),
        pltpu.VMEM((_CPW,), f32),
        pltpu.VMEM((_CPW,), f32),
        pltpu.VMEM((_CPW,), f32),
        pltpu.VMEM((_CPW, W1), f32),
        pltpu.VMEM((_CPW, W2), f32),
        pltpu.SemaphoreType.DMA,
        pltpu.SemaphoreType.DMA,
    ]
    fn = pl.kernel(_sc_body,
                   out_type=(sds((N_CUTS,), f32), sds((N_CUTS,), f32)),
                   mesh=mesh, scratch_types=scratch,
                   compiler_params=pltpu.CompilerParams(
                       needs_layout_passes=False))
    return fn(x, rxg, t1, t2)


def kernel(cut_positions, cut_local_reflatentxgene_ix, cut_local_gene_ix,
           cut_local_reflatent_ix, mixture_delta_reflatentxgene,
           unnormalized_heights, unnormalized_widths):
    del cut_local_gene_ix, cut_local_reflatent_ix
    t1, t2 = _compute_tables(unnormalized_widths, unnormalized_heights,
                             mixture_delta_reflatentxgene)
    t1 = t1.reshape(N_REFLATENT * N_GENES, W1)
    t2 = t2.reshape(N_REFLATENT * N_GENES, W2)
    rxg = cut_local_reflatentxgene_ix.astype(jnp.int32)
    out, logabsdet = _sc_transform(cut_positions, rxg, t1, t2)
    return out, logabsdet


# speedup vs baseline: 18.4991x; 1.0076x over previous
import jax
import jax.numpy as jnp
from jax import lax
from jax.experimental import pallas as pl
from jax.experimental.pallas import tpu as pltpu
from jax.experimental.pallas import tpu_sc as plsc

N_GENES = 128
N_REFLATENT = 16
N_CUTS = 2048
B1 = 64
B2 = 32
W1 = 4 * B1
W2 = 4 * B2

_NC = 2
_NS = 16
_NW = _NC * _NS
_CPW = N_CUTS // _NW

_HIGH = lax.Precision.HIGHEST


def _strict_lower(n):
    ii = lax.broadcasted_iota(jnp.int32, (n, n), 0)
    jj = lax.broadcasted_iota(jnp.int32, (n, n), 1)
    return (ii < jj).astype(jnp.float32)


def _shift_down_genes(x, k):
    pad = jnp.zeros((N_REFLATENT, k, 1), jnp.float32)
    return jnp.concatenate([pad, x[:, : N_GENES - k, :]], axis=1)


def _excl_cumsum_genes(s):
    incl = s
    k = 1
    while k < N_GENES:
        incl = incl + _shift_down_genes(incl, k)
        k *= 2
    return incl - s


def _tables_body(uw_ref, uh_ref, d_ref, o1_ref, o2_ref):
    f32 = jnp.float32
    M64 = _strict_lower(B1)
    M32 = _strict_lower(B2)
    MG = _strict_lower(N_GENES)
    neg = jnp.full((N_GENES, 1), -1e9, f32)
    uw1 = jnp.concatenate([uw_ref[:, : B1 - 1], neg], axis=-1)
    uw2 = jnp.concatenate([uw_ref[:, B1 - 1:], neg], axis=-1)
    uh1 = uh_ref[:, :B1]
    uh2 = uh_ref[:, B1:]
    d1 = d_ref[:, :, :B1]
    d2 = d_ref[:, :, B1:]

    def transform(uw, uh, d, gspace, nb, M, o_ref):
        e = jnp.exp(uw)
        sm = e / jnp.sum(e, axis=-1, keepdims=True)
        w3 = sm[None, :, :] * gspace
        cum_in = jnp.dot(w3.reshape(N_REFLATENT * N_GENES, nb), M,
                         precision=_HIGH).reshape(N_REFLATENT, N_GENES, nb)
        s_w = jnp.sum(w3, axis=-1, keepdims=True)
        offs_w = _excl_cumsum_genes(s_w)
        gg = lax.broadcasted_iota(jnp.int32, (N_REFLATENT, N_GENES, nb), 1)
        bb = lax.broadcasted_iota(jnp.int32, (N_REFLATENT, N_GENES, nb), 2)
        last = jnp.logical_and(gg == N_GENES - 1, bb == nb - 1)
        locs = jnp.where(last, f32(1.0), offs_w + cum_in)
        o_ref[:, :, 0:nb] = locs
        o_ref[:, :, nb:2 * nb] = w3
        h = jnp.exp(d + uh[None, :, :])
        hn = jnp.concatenate([h[:, :, 1:], h[:, :, nb - 1:]], axis=-1)
        c = (h + hn) * 0.5 * w3
        s_c = jnp.sum(c, axis=-1, keepdims=True)
        area = jnp.sum(s_c, axis=1, keepdims=True)
        inv_area = 1.0 / area
        o_ref[:, :, 2 * nb:3 * nb] = h * inv_area
        cn = c * inv_area
        cdf_in = jnp.dot(cn.reshape(N_REFLATENT * N_GENES, nb), M,
                         precision=_HIGH).reshape(N_REFLATENT, N_GENES, nb)
        s_cn = s_c * inv_area
        offs_c = _excl_cumsum_genes(s_cn)
        cdf = jnp.where(last, f32(1.0), offs_c + cdf_in)
        o_ref[:, :, 3 * nb:4 * nb] = cdf
        gs = offs_c + s_cn
        gg1 = lax.broadcasted_iota(jnp.int32, (N_REFLATENT, N_GENES, 1), 1)
        gs = jnp.where(gg1 == N_GENES - 1, f32(1.0), gs)
        return gs - _shift_down_genes(gs, 1)

    gs1 = jnp.full((N_REFLATENT, N_GENES, 1), 1.0 / N_GENES, jnp.float32)
    gs2 = transform(uw1, uh1, d1, gs1, B1, M64, o1_ref)
    transform(uw2, uh2, d2, gs2, B2, M32, o2_ref)


def _compute_tables(uw, uh, d):
    f32 = jnp.float32
    sds = jax.ShapeDtypeStruct
    out_shape = [
        sds((N_REFLATENT, N_GENES, W1), f32),
        sds((N_REFLATENT, N_GENES, W2), f32),
    ]
    return pl.pallas_call(_tables_body, out_shape=out_shape)(uw, uh, d)


def _spline_batch(tab, rows, x, nb):
    i32 = jnp.int32
    lo = jnp.zeros((16,), i32)
    step = nb // 2
    while step >= 1:
        edge = plsc.load_gather(tab, [rows, lo + (step - 1)])
        lo = lo + jnp.where(edge < x, i32(step), i32(0))
        step //= 2
    jr = jnp.minimum(jnp.maximum(lo - 1, i32(0)), i32(nb - 2))
    ll = plsc.load_gather(tab, [rows, jr])
    w = plsc.load_gather(tab, [rows, nb + jr])
    hl = plsc.load_gather(tab, [rows, 2 * nb + jr])
    hr = plsc.load_gather(tab, [rows, 2 * nb + 1 + jr])
    cl = plsc.load_gather(tab, [rows, 3 * nb + jr])
    alpha = (x - ll) / w
    dh = hr - hl
    out = (0.5 * dh * w) * alpha * alpha + (hl * w) * alpha + cl
    out = jnp.minimum(jnp.maximum(out, 0.0), 1.0)
    hval = hl + alpha * dh
    return out, hval


def _ln16(x):
    i32 = jnp.int32
    bits = plsc.bitcast(x, jnp.int32)
    e = lax.shift_right_logical(bits, 23) - 127
    m = plsc.bitcast((bits & i32(0x7FFFFF)) | i32(127 << 23), jnp.float32)
    big = m > 1.4142135
    m = jnp.where(big, m * 0.5, m)
    e = e + jnp.where(big, i32(1), i32(0))
    r = (m - 1.0) / (m + 1.0)
    r2 = r * r
    p = 2.0 * r * (1.0 + r2 * (1.0 / 3.0 + r2 * (0.2 + r2 * (1.0 / 7.0))))
    return e.astype(jnp.float32) * 0.6931471805599453 + p


def _sc_body(x_hbm, rxg_hbm, t1, t2, out_hbm, lad_hbm,
             idxv, xbuf, hp, outb, tab1, tab2, sem1, sem2):
    wid = lax.axis_index("s") * _NC + lax.axis_index("c")
    base = wid * _CPW
    pltpu.sync_copy(rxg_hbm.at[pl.ds(base, _CPW)], idxv)
    cp1 = pltpu.async_copy(t1.at[idxv], tab1, sem1)
    cp2 = pltpu.async_copy(t2.at[idxv], tab2, sem2)
    pltpu.sync_copy(x_hbm.at[pl.ds(base, _CPW)], xbuf)
    cp1.wait()
    iota = lax.broadcasted_iota(jnp.int32, (16,), 0)
    for v in range(_CPW // 16):
        sl = pl.ds(v * 16, 16)
        rows = iota + (v * 16)
        out, hval = _spline_batch(tab1, rows, xbuf[sl], B1)
        xbuf[sl] = out
        hp[sl] = hval
    cp2.wait()
    for v in range(_CPW // 16):
        sl = pl.ds(v * 16, 16)
        rows = iota + (v * 16)
        out, hval = _spline_batch(tab2, rows, xbuf[sl], B2)
        outb[sl] = out
        hp[sl] = _ln16(hp[sl] * hval)
    pltpu.sync_copy(outb, out_hbm.at[pl.ds(base, _CPW)])
    pltpu.sync_copy(hp, lad_hbm.at[pl.ds(base, _CPW)])


def _sc_transform(x, rxg, t1, t2):
    f32 = jnp.float32
    sds = jax.ShapeDtypeStruct
    mesh = plsc.VectorSubcoreMesh(core_axis_name="c", subcore_axis_name="s",
                                  num_cores=_NC, num_subcores=_NS)
    scratch = [
        pltpu.VMEM((_CPW,), jnp.int32),
        pltpu.VMEM((_CPW,), f32),
        pltpu.VMEM((_CPW,), f32),
        pltpu.VMEM((_CPW,), f32),
        pltpu.VMEM((_CPW, W1), f32),
        pltpu.VMEM((_CPW, W2), f32),
        pltpu.SemaphoreType.DMA,
        pltpu.SemaphoreType.DMA,
    ]
    fn = pl.kernel(_sc_body,
                   out_type=(sds((N_CUTS,), f32), sds((N_CUTS,), f32)),
                   mesh=mesh, scratch_types=scratch,
                   compiler_params=pltpu.CompilerParams(
                       needs_layout_passes=False))
    return fn(x, rxg, t1, t2)


def kernel(cut_positions, cut_local_reflatentxgene_ix, cut_local_gene_ix,
           cut_local_reflatent_ix, mixture_delta_reflatentxgene,
           unnormalized_heights, unnormalized_widths):
    del cut_local_gene_ix, cut_local_reflatent_ix
    t1, t2 = _compute_tables(unnormalized_widths, unnormalized_heights,
                             mixture_delta_reflatentxgene)
    t1 = t1.reshape(N_REFLATENT * N_GENES, W1)
    t2 = t2.reshape(N_REFLATENT * N_GENES, W2)
    rxg = cut_local_reflatentxgene_ix.astype(jnp.int32)
    out, logabsdet = _sc_transform(cut_positions, rxg, t1, t2)
    return out, logabsdet


# speedup vs baseline: 18.5126x; 1.0007x over previous
import jax
import jax.numpy as jnp
from jax import lax
from jax.experimental import pallas as pl
from jax.experimental.pallas import tpu as pltpu
from jax.experimental.pallas import tpu_sc as plsc

N_GENES = 128
N_REFLATENT = 16
N_CUTS = 2048
B1 = 64
B2 = 32
W1 = 4 * B1
W2 = 4 * B2

_NC = 2
_NS = 16
_NW = _NC * _NS
_CPW = N_CUTS // _NW

_HIGH = lax.Precision.HIGHEST


def _strict_lower(n):
    ii = lax.broadcasted_iota(jnp.int32, (n, n), 0)
    jj = lax.broadcasted_iota(jnp.int32, (n, n), 1)
    return (ii < jj).astype(jnp.float32)


def _shift_down_genes(x, k):
    pad = jnp.zeros((N_REFLATENT, k, 1), jnp.float32)
    return jnp.concatenate([pad, x[:, : N_GENES - k, :]], axis=1)


def _excl_cumsum_genes(s):
    incl = s
    k = 1
    while k < N_GENES:
        incl = incl + _shift_down_genes(incl, k)
        k *= 2
    return incl - s


def _tables_body(uw_ref, uh_ref, d_ref, o1_ref, o2_ref):
    f32 = jnp.float32
    M64 = _strict_lower(B1)
    M32 = _strict_lower(B2)
    neg = jnp.full((N_GENES, 1), -1e9, f32)
    uw1 = jnp.concatenate([uw_ref[:, : B1 - 1], neg], axis=-1)
    uw2 = jnp.concatenate([uw_ref[:, B1 - 1:], neg], axis=-1)
    uh1 = uh_ref[:, :B1]
    uh2 = uh_ref[:, B1:]
    d1 = d_ref[:, :, :B1]
    d2 = d_ref[:, :, B1:]

    def transform(uw, uh, d, gspace, nb, M, o_ref):
        e = jnp.exp(uw)
        sm = e / jnp.sum(e, axis=-1, keepdims=True)
        w3 = sm[None, :, :] * gspace
        cum_in = jnp.dot(w3.reshape(N_REFLATENT * N_GENES, nb), M,
                         precision=_HIGH).reshape(N_REFLATENT, N_GENES, nb)
        s_w = jnp.sum(w3, axis=-1, keepdims=True)
        offs_w = _excl_cumsum_genes(s_w)
        gg = lax.broadcasted_iota(jnp.int32, (N_REFLATENT, N_GENES, nb), 1)
        bb = lax.broadcasted_iota(jnp.int32, (N_REFLATENT, N_GENES, nb), 2)
        last = jnp.logical_and(gg == N_GENES - 1, bb == nb - 1)
        locs = jnp.where(last, f32(1.0), offs_w + cum_in)
        o_ref[:, :, 0:nb] = locs
        o_ref[:, :, nb:2 * nb] = w3
        h = jnp.exp(d + uh[None, :, :])
        hn = jnp.concatenate([h[:, :, 1:], h[:, :, nb - 1:]], axis=-1)
        c = (h + hn) * 0.5 * w3
        s_c = jnp.sum(c, axis=-1, keepdims=True)
        area = jnp.sum(s_c, axis=1, keepdims=True)
        inv_area = 1.0 / area
        o_ref[:, :, 2 * nb:3 * nb] = h * inv_area
        cn = c * inv_area
        cdf_in = jnp.dot(cn.reshape(N_REFLATENT * N_GENES, nb), M,
                         precision=_HIGH).reshape(N_REFLATENT, N_GENES, nb)
        s_cn = s_c * inv_area
        offs_c = _excl_cumsum_genes(s_cn)
        cdf = jnp.where(last, f32(1.0), offs_c + cdf_in)
        o_ref[:, :, 3 * nb:4 * nb] = cdf
        gs = offs_c + s_cn
        gg1 = lax.broadcasted_iota(jnp.int32, (N_REFLATENT, N_GENES, 1), 1)
        gs = jnp.where(gg1 == N_GENES - 1, f32(1.0), gs)
        return gs - _shift_down_genes(gs, 1)

    gs1 = jnp.full((N_REFLATENT, N_GENES, 1), 1.0 / N_GENES, jnp.float32)
    gs2 = transform(uw1, uh1, d1, gs1, B1, M64, o1_ref)
    transform(uw2, uh2, d2, gs2, B2, M32, o2_ref)


def _compute_tables(uw, uh, d):
    f32 = jnp.float32
    sds = jax.ShapeDtypeStruct
    out_shape = [
        sds((N_REFLATENT, N_GENES, W1), f32),
        sds((N_REFLATENT, N_GENES, W2), f32),
    ]
    return pl.pallas_call(_tables_body, out_shape=out_shape)(uw, uh, d)


def _spline_batch(tab, rows, x, nb):
    i32 = jnp.int32
    lo = jnp.zeros((16,), i32)
    step = nb // 2
    while step >= 1:
        edge = plsc.load_gather(tab, [rows, lo + (step - 1)])
        lo = lo + jnp.where(edge < x, i32(step), i32(0))
        step //= 2
    jr = jnp.minimum(jnp.maximum(lo - 1, i32(0)), i32(nb - 2))
    ll = plsc.load_gather(tab, [rows, jr])
    w = plsc.load_gather(tab, [rows, nb + jr])
    hl = plsc.load_gather(tab, [rows, 2 * nb + jr])
    hr = plsc.load_gather(tab, [rows, 2 * nb + 1 + jr])
    cl = plsc.load_gather(tab, [rows, 3 * nb + jr])
    alpha = (x - ll) / w
    dh = hr - hl
    out = (0.5 * dh * w) * alpha * alpha + (hl * w) * alpha + cl
    out = jnp.minimum(jnp.maximum(out, 0.0), 1.0)
    hval = hl + alpha * dh
    return out, hval


def _ln16(x):
    i32 = jnp.int32
    bits = plsc.bitcast(x, jnp.int32)
    e = lax.shift_right_logical(bits, 23) - 127
    m = plsc.bitcast((bits & i32(0x7FFFFF)) | i32(127 << 23), jnp.float32)
    big = m > 1.4142135
    m = jnp.where(big, m * 0.5, m)
    e = e + jnp.where(big, i32(1), i32(0))
    r = (m - 1.0) / (m + 1.0)
    r2 = r * r
    p = 2.0 * r * (1.0 + r2 * (1.0 / 3.0 + r2 * (0.2 + r2 * (1.0 / 7.0))))
    return e.astype(jnp.float32) * 0.6931471805599453 + p


def _sc_body(x_hbm, rxg_hbm, t1, t2, out_hbm, lad_hbm,
             idxv, xbuf, hp, outb, tab1, tab2, sem1, sem2):
    wid = lax.axis_index("s") * _NC + lax.axis_index("c")
    base = wid * _CPW
    pltpu.sync_copy(rxg_hbm.at[pl.ds(base, _CPW)], idxv)
    cp1 = pltpu.async_copy(t1.at[idxv], tab1, sem1)
    cp2 = pltpu.async_copy(t2.at[idxv], tab2, sem2)
    pltpu.sync_copy(x_hbm.at[pl.ds(base, _CPW)], xbuf)
    cp1.wait()
    iota = lax.broadcasted_iota(jnp.int32, (16,), 0)
    for v in range(_CPW // 16):
        sl = pl.ds(v * 16, 16)
        rows = iota + (v * 16)
        out, hval = _spline_batch(tab1, rows, xbuf[sl], B1)
        xbuf[sl] = out
        hp[sl] = hval
    cp2.wait()
    for v in range(_CPW // 16):
        sl = pl.ds(v * 16, 16)
        rows = iota + (v * 16)
        out, hval = _spline_batch(tab2, rows, xbuf[sl], B2)
        outb[sl] = out
        hp[sl] = _ln16(hp[sl] * hval)
    pltpu.sync_copy(outb, out_hbm.at[pl.ds(base, _CPW)])
    pltpu.sync_copy(hp, lad_hbm.at[pl.ds(base, _CPW)])


def _sc_transform(x, rxg, t1, t2):
    f32 = jnp.float32
    sds = jax.ShapeDtypeStruct
    mesh = plsc.VectorSubcoreMesh(core_axis_name="c", subcore_axis_name="s",
                                  num_cores=_NC, num_subcores=_NS)
    scratch = [
        pltpu.VMEM((_CPW,), jnp.int32),
        pltpu.VMEM((_CPW,), f32),
        pltpu.VMEM((_CPW,), f32),
        pltpu.VMEM((_CPW,), f32),
        pltpu.VMEM((_CPW, W1), f32),
        pltpu.VMEM((_CPW, W2), f32),
        pltpu.SemaphoreType.DMA,
        pltpu.SemaphoreType.DMA,
    ]
    fn = pl.kernel(_sc_body,
                   out_type=(sds((N_CUTS,), f32), sds((N_CUTS,), f32)),
                   mesh=mesh, scratch_types=scratch,
                   compiler_params=pltpu.CompilerParams(
                       needs_layout_passes=False))
    return fn(x, rxg, t1, t2)


def kernel(cut_positions, cut_local_reflatentxgene_ix, cut_local_gene_ix,
           cut_local_reflatent_ix, mixture_delta_reflatentxgene,
           unnormalized_heights, unnormalized_widths):
    del cut_local_gene_ix, cut_local_reflatent_ix
    t1, t2 = _compute_tables(unnormalized_widths, unnormalized_heights,
                             mixture_delta_reflatentxgene)
    t1 = t1.reshape(N_REFLATENT * N_GENES, W1)
    t2 = t2.reshape(N_REFLATENT * N_GENES, W2)
    rxg = cut_local_reflatentxgene_ix.astype(jnp.int32)
    out, logabsdet = _sc_transform(cut_positions, rxg, t1, t2)
    return out, logabsdet
